# Initial kernel scaffold; baseline (speedup 1.0000x reference)
import jax
import jax.numpy as jnp
from jax.experimental import pallas as pl


def kernel(output, nms_tresh):
    raise NotImplementedError("write your pallas kernel here")


# speedup vs baseline: 9.3423x; 9.3423x over previous
import functools

import jax
import jax.numpy as jnp
from jax.experimental import pallas as pl
from jax.experimental.pallas import tpu as pltpu

_ANCHORS = [12.0, 16.0, 19.0, 36.0, 40.0, 28.0, 36.0, 75.0, 76.0, 55.0,
            72.0, 146.0, 142.0, 110.0, 192.0, 243.0, 459.0, 401.0]
_ANCHOR_MASK = [6, 7, 8]
_NUM_CLASSES = 80
_STRIDE = 32


def _sigmoid(x):
    return 1.0 / (1.0 + jnp.exp(-x))


def _yolo_body(thresh_ref, x_ref, boxes_ref, keep_ref, *, H, W, aw, ah):
    a = pl.program_id(1)
    t = x_ref[0, 0]
    n = H * W

    idx = jax.lax.broadcasted_iota(jnp.int32, (1, n), 1)
    gx = (idx % W).astype(jnp.float32)
    gy = (idx // W).astype(jnp.float32)

    xs = (_sigmoid(t[0:1, :]) + gx) * (1.0 / W)
    ys = (_sigmoid(t[1:2, :]) + gy) * (1.0 / H)

    aw_c = jnp.where(a == 0, aw[0], jnp.where(a == 1, aw[1], aw[2]))
    ah_c = jnp.where(a == 0, ah[0], jnp.where(a == 1, ah[1], ah[2]))
    ws = jnp.exp(t[2:3, :]) * aw_c
    hs = jnp.exp(t[3:4, :]) * ah_c

    det = _sigmoid(t[4:5, :])

    cls = t[5:5 + _NUM_CLASSES, :]
    m = jnp.max(cls, axis=0, keepdims=True)
    s = jnp.sum(jnp.exp(cls - m), axis=0, keepdims=True)
    conf = 1.0 / s
    cidx = jax.lax.broadcasted_iota(jnp.int32, cls.shape, 0)
    first_max = jnp.min(
        jnp.where(cls == m, cidx, _NUM_CLASSES), axis=0, keepdims=True)
    cid = first_max.astype(jnp.float32)

    boxes_ref[0, 0] = jnp.concatenate([xs, ys, ws, hs, det, conf, cid], axis=0)
    keep_ref[0, 0] = det > thresh_ref[0]


def kernel(output, nms_tresh):
    B, C, H, W = output.shape
    A = len(_ANCHOR_MASK)
    n = H * W
    x = output.reshape(B, A, C // A, n)
    th = jnp.asarray(nms_tresh, jnp.float32).reshape(1)

    aw = tuple(_ANCHORS[m * 2] / _STRIDE / W for m in _ANCHOR_MASK)
    ah = tuple(_ANCHORS[m * 2 + 1] / _STRIDE / H for m in _ANCHOR_MASK)

    body = functools.partial(_yolo_body, H=H, W=W, aw=aw, ah=ah)
    boxes_t, keep4 = pl.pallas_call(
        body,
        grid=(B, A),
        in_specs=[
            pl.BlockSpec(memory_space=pltpu.SMEM),
            pl.BlockSpec((1, 1, C // A, n), lambda b, a: (b, a, 0, 0)),
        ],
        out_specs=[
            pl.BlockSpec((1, 1, 7, n), lambda b, a: (b, a, 0, 0)),
            pl.BlockSpec((1, 1, 1, n), lambda b, a: (b, a, 0, 0)),
        ],
        out_shape=[
            jax.ShapeDtypeStruct((B, A, 7, n), jnp.float32),
            jax.ShapeDtypeStruct((B, A, 1, n), jnp.bool_),
        ],
    )(th, x)

    boxes = boxes_t.transpose(0, 1, 3, 2).reshape(B, A * n, 7)
    keep = keep4.reshape(B, A * n)
    return boxes, keep


# speedup vs baseline: 11.7375x; 1.2564x over previous
import functools

import jax
import jax.numpy as jnp
from jax.experimental import pallas as pl
from jax.experimental.pallas import tpu as pltpu

_ANCHORS = [12.0, 16.0, 19.0, 36.0, 40.0, 28.0, 36.0, 75.0, 76.0, 55.0,
            72.0, 146.0, 142.0, 110.0, 192.0, 243.0, 459.0, 401.0]
_ANCHOR_MASK = [6, 7, 8]
_NUM_CLASSES = 80
_STRIDE = 32
_BB = 8


def _sigmoid(x):
    return 1.0 / (1.0 + jnp.exp(-x))


def _yolo_body(thresh_ref, x_ref, boxes_ref, keep_ref, *, H, W, aw, ah, bb):
    n = H * W
    A = len(aw)
    idx = jax.lax.broadcasted_iota(jnp.int32, (1, n), 1)
    gx = (idx % W).astype(jnp.float32)
    gy = (idx // W).astype(jnp.float32)
    th = thresh_ref[0]

    for i in range(bb):
        for a in range(A):
            t = x_ref[i, a]
            xs = (_sigmoid(t[0:1, :]) + gx) * (1.0 / W)
            ys = (_sigmoid(t[1:2, :]) + gy) * (1.0 / H)
            ws = jnp.exp(t[2:3, :]) * aw[a]
            hs = jnp.exp(t[3:4, :]) * ah[a]
            det = _sigmoid(t[4:5, :])

            cls = t[5:5 + _NUM_CLASSES, :]
            m = jnp.max(cls, axis=0, keepdims=True)
            s = jnp.sum(jnp.exp(cls - m), axis=0, keepdims=True)
            conf = 1.0 / s
            cidx = jax.lax.broadcasted_iota(jnp.int32, cls.shape, 0)
            first_max = jnp.min(
                jnp.where(cls == m, cidx, _NUM_CLASSES), axis=0, keepdims=True)
            cid = first_max.astype(jnp.float32)

            boxes_ref[i, a] = jnp.concatenate(
                [xs, ys, ws, hs, det, conf, cid], axis=0)
            keep_ref[i, a] = det > th


def kernel(output, nms_tresh):
    B, C, H, W = output.shape
    A = len(_ANCHOR_MASK)
    n = H * W
    x = output.reshape(B, A, C // A, n)
    th = jnp.asarray(nms_tresh, jnp.float32).reshape(1)

    aw = tuple(_ANCHORS[m * 2] / _STRIDE / W for m in _ANCHOR_MASK)
    ah = tuple(_ANCHORS[m * 2 + 1] / _STRIDE / H for m in _ANCHOR_MASK)

    bb = _BB if B % _BB == 0 else 1
    body = functools.partial(_yolo_body, H=H, W=W, aw=aw, ah=ah, bb=bb)
    boxes_t, keep4 = pl.pallas_call(
        body,
        grid=(B // bb,),
        in_specs=[
            pl.BlockSpec(memory_space=pltpu.SMEM),
            pl.BlockSpec((bb, A, C // A, n), lambda b: (b, 0, 0, 0)),
        ],
        out_specs=[
            pl.BlockSpec((bb, A, 7, n), lambda b: (b, 0, 0, 0)),
            pl.BlockSpec((bb, A, 1, n), lambda b: (b, 0, 0, 0)),
        ],
        out_shape=[
            jax.ShapeDtypeStruct((B, A, 7, n), jnp.float32),
            jax.ShapeDtypeStruct((B, A, 1, n), jnp.bool_),
        ],
    )(th, x)

    boxes = boxes_t.transpose(0, 1, 3, 2).reshape(B, A * n, 7)
    keep = keep4.reshape(B, A * n)
    return boxes, keep
